# baseline (speedup 1.0000x reference)
import functools

import jax
import jax.numpy as jnp
from jax import lax
from jax.experimental import pallas as pl
from jax.experimental.pallas import tpu as pltpu
from jax.experimental.pallas import tpu_sc as plsc

HIDDEN = 1024
FFN = 2048
MOE_FFN = 512
E = 8
TOP_K = 2
T = 2048
A = T * TOP_K
BLK = 128
NPAD = A + E * BLK
NBLK = NPAD // BLK
TOK_BLK = 256

NW = 16
APT = A // NW


def _router_body(x_ref, wr_ref, tk_ref, w_ref):
    logits = jnp.dot(x_ref[...], wr_ref[...], preferred_element_type=jnp.float32)
    iota = lax.broadcasted_iota(jnp.int32, logits.shape, 1)
    v1 = jnp.max(logits, axis=1, keepdims=True)
    idx1 = jnp.min(jnp.where(logits == v1, iota, E), axis=1, keepdims=True)
    sel1 = iota == idx1
    masked = jnp.where(sel1, -jnp.inf, logits)
    v2 = jnp.max(masked, axis=1, keepdims=True)
    idx2 = jnp.min(jnp.where(masked == v2, iota, E), axis=1, keepdims=True)
    e2 = jnp.exp(v2 - v1)
    w1 = 1.0 / (1.0 + e2)
    w2 = 1.0 - w1
    tk_ref[...] = jnp.concatenate([idx1, idx2], axis=1)
    w_ref[...] = jnp.concatenate([w1, w2], axis=1)


def _router(x, wr):
    n = T // TOK_BLK
    return pl.pallas_call(
        _router_body,
        grid=(n,),
        in_specs=[
            pl.BlockSpec((TOK_BLK, HIDDEN), lambda t: (t, 0)),
            pl.BlockSpec((HIDDEN, E), lambda t: (0, 0)),
        ],
        out_specs=[
            pl.BlockSpec((TOK_BLK, TOP_K), lambda t: (t, 0)),
            pl.BlockSpec((TOK_BLK, TOP_K), lambda t: (t, 0)),
        ],
        out_shape=[
            jax.ShapeDtypeStruct((T, TOP_K), jnp.int32),
            jax.ShapeDtypeStruct((T, TOP_K), jnp.float32),
        ],
    )(x, wr)


def _shared_body(x_ref, wg_ref, wu_ref, wd_ref, shared_ref):
    xb = x_ref[...].astype(jnp.bfloat16)
    g = jnp.dot(xb, wg_ref[...], preferred_element_type=jnp.float32)
    u = jnp.dot(xb, wu_ref[...], preferred_element_type=jnp.float32)
    h = (g * jax.nn.sigmoid(g) * u).astype(jnp.bfloat16)
    shared_ref[...] = jnp.dot(h, wd_ref[...], preferred_element_type=jnp.float32)


def _shared(x, wg, wu, wd):
    n = T // TOK_BLK
    return pl.pallas_call(
        _shared_body,
        grid=(n,),
        in_specs=[
            pl.BlockSpec((TOK_BLK, HIDDEN), lambda t: (t, 0)),
            pl.BlockSpec((HIDDEN, FFN), lambda t: (0, 0)),
            pl.BlockSpec((HIDDEN, FFN), lambda t: (0, 0)),
            pl.BlockSpec((FFN, HIDDEN), lambda t: (0, 0)),
        ],
        out_specs=pl.BlockSpec((TOK_BLK, HIDDEN), lambda t: (t, 0)),
        out_shape=jax.ShapeDtypeStruct((T, HIDDEN), jnp.float32),
    )(x, wg, wu, wd)


NW2 = 32
APT2 = A // NW2
VPT = APT2 // 16
NVREG = A // 16


def _routing_sc(x, tkT):
    mesh = plsc.VectorSubcoreMesh(
        core_axis_name="c", subcore_axis_name="s", num_cores=2)

    @functools.partial(
        pl.kernel,
        out_type=[
            jax.ShapeDtypeStruct((NPAD, HIDDEN), jnp.float32),
            jax.ShapeDtypeStruct((A,), jnp.int32),
            jax.ShapeDtypeStruct((48,), jnp.int32),
        ],
        mesh=mesh,
        scratch_types=[
            pltpu.VMEM((A,), jnp.int32),
            pltpu.VMEM((2, 64), jnp.int32),
            pltpu.VMEM((64, HIDDEN), jnp.float32),
            pltpu.VMEM((16,), jnp.int32),
            pltpu.SemaphoreType.DMA,
        ],
        compiler_params=pltpu.CompilerParams(needs_layout_passes=False),
    )
    def routing(xb_hbm, tkT_hbm, sx_hbm, ip_hbm, be_hbm,
                eids_v, pos_v, rows_v, stage_v, sem):
        wid = lax.axis_index("s") * 2 + lax.axis_index("c")
        iota = lax.iota(jnp.int32, 16)

        pltpu.sync_copy(tkT_hbm, eids_v)

        def scan_body(jj, carry):
            cnt, pre = carry
            pre = jnp.where(jj == wid * VPT, cnt, pre)
            v = eids_v[pl.ds(jj * 16, 16)]
            for e in range(E):
                ce = plsc.all_reduce_population_count(v == e)
                cnt = cnt + jnp.where(iota == e, ce, 0)
            return cnt, pre

        cnt, pre = lax.fori_loop(
            0, NVREG, scan_body,
            (jnp.zeros(16, jnp.int32), jnp.zeros(16, jnp.int32)))
        tot = cnt
        padded = ((tot + (BLK - 1)) >> 7) << 7
        pstart = plsc.cumsum(padded) - padded
        base = pstart + pre

        @pl.when(wid == 0)
        def _():
            sblk = pstart >> 7
            eblk = sblk + (padded >> 7)
            for bj in range(3):
                bvec = bj * 16 + iota
                be = jnp.zeros(16, jnp.int32)
                for e in range(1, E):
                    s_e = jnp.sum(jnp.where(iota == e, sblk, 0))
                    e_e = jnp.sum(jnp.where(iota == e, eblk, 0))
                    be = be + jnp.where((bvec >= s_e) & (bvec < e_e), e, 0)
                stage_v[...] = be
                pltpu.sync_copy(stage_v, be_hbm.at[pl.ds(bj * 16, 16)])

        k_tok0 = lax.rem(wid, NW2 // TOP_K) * APT2
        for q in range(2):
            pos_ref = pos_v.at[q]
            for j in range(4):
                v = eids_v[pl.ds(wid * APT2 + q * 64 + j * 16, 16)]
                pos = jnp.zeros(16, jnp.int32)
                for e in range(E):
                    m = v == e
                    mi = jnp.where(m, 1, 0)
                    excl = plsc.cumsum(mi) - mi
                    b_e = jnp.sum(jnp.where(iota == e, base, 0))
                    pos = pos + mi * (b_e + excl)
                    c_e = plsc.all_reduce_population_count(m)
                    base = base + jnp.where(iota == e, c_e, 0)
                pos_ref[pl.ds(j * 16, 16)] = pos
            pltpu.sync_copy(pos_ref, ip_hbm.at[pl.ds(wid * APT2 + q * 64, 64)])
            pltpu.sync_copy(xb_hbm.at[pl.ds(k_tok0 + q * 64, 64)], rows_v)
            pltpu.async_copy(rows_v, sx_hbm.at[pos_ref], sem).wait()

    return routing(x, tkT)


def _gmm_body(be_ref, xs_ref, wg_ref, wu_ref, wd_ref, y_ref):
    xb = xs_ref[...].astype(jnp.bfloat16)
    g = jnp.dot(xb, wg_ref[0], preferred_element_type=jnp.float32)
    u = jnp.dot(xb, wu_ref[0], preferred_element_type=jnp.float32)
    h = (g * jax.nn.sigmoid(g) * u).astype(jnp.bfloat16)
    y_ref[...] = jnp.dot(h, wd_ref[0], preferred_element_type=jnp.float32)


def _gmm(blk_exp, sorted_x, weg, weu, wed):
    grid_spec = pltpu.PrefetchScalarGridSpec(
        num_scalar_prefetch=1,
        grid=(NBLK,),
        in_specs=[
            pl.BlockSpec((BLK, HIDDEN), lambda b, be: (b, 0)),
            pl.BlockSpec((1, HIDDEN, MOE_FFN), lambda b, be: (be[b], 0, 0)),
            pl.BlockSpec((1, HIDDEN, MOE_FFN), lambda b, be: (be[b], 0, 0)),
            pl.BlockSpec((1, MOE_FFN, HIDDEN), lambda b, be: (be[b], 0, 0)),
        ],
        out_specs=pl.BlockSpec((BLK, HIDDEN), lambda b, be: (b, 0)),
    )
    return pl.pallas_call(
        _gmm_body,
        grid_spec=grid_spec,
        out_shape=jax.ShapeDtypeStruct((NPAD, HIDDEN), jnp.float32),
    )(blk_exp, sorted_x, weg, weu, wed)


def _gather_sc(y_out, ip):
    mesh = plsc.VectorSubcoreMesh(
        core_axis_name="c", subcore_axis_name="s", num_cores=2)

    @functools.partial(
        pl.kernel,
        out_type=jax.ShapeDtypeStruct((A, HIDDEN), jnp.float32),
        mesh=mesh,
        scratch_types=[
            pltpu.VMEM((64,), jnp.int32),
            pltpu.VMEM((64, HIDDEN), jnp.float32),
            pltpu.SemaphoreType.DMA,
        ],
        compiler_params=pltpu.CompilerParams(needs_layout_passes=False),
    )
    def gatherk(y_hbm, ip_hbm, yg_hbm, idx_v, gbuf_v, sem):
        wid = lax.axis_index("s") * 2 + lax.axis_index("c")
        for half in range(2):
            base = wid * BLK + half * 64
            pltpu.sync_copy(ip_hbm.at[pl.ds(base, 64)], idx_v)
            pltpu.async_copy(y_hbm.at[idx_v], gbuf_v, sem).wait()
            pltpu.sync_copy(gbuf_v, yg_hbm.at[pl.ds(base, 64)])

    return gatherk(y_out, ip)


def _combine_body(sh_ref, w_ref, y0_ref, y1_ref, out_ref):
    w = w_ref[...]
    out_ref[...] = (sh_ref[...]
                    + w[:, 0:1] * y0_ref[...]
                    + w[:, 1:2] * y1_ref[...])


def _combine(shared, w2, yg):
    n = T // TOK_BLK
    return pl.pallas_call(
        _combine_body,
        grid=(n,),
        in_specs=[
            pl.BlockSpec((TOK_BLK, HIDDEN), lambda t: (t, 0)),
            pl.BlockSpec((TOK_BLK, TOP_K), lambda t: (t, 0)),
            pl.BlockSpec((TOK_BLK, HIDDEN), lambda t: (t, 0)),
            pl.BlockSpec((TOK_BLK, HIDDEN), lambda t, n=n: (t + n, 0)),
        ],
        out_specs=pl.BlockSpec((TOK_BLK, HIDDEN), lambda t: (t, 0)),
        out_shape=jax.ShapeDtypeStruct((T, HIDDEN), jnp.float32),
    )(shared, w2, yg, yg)


def kernel(hidden_states, W_router, Ws_gate, Ws_up, Ws_down,
           We_gate, We_up, We_down):
    orig_shape = hidden_states.shape
    x = hidden_states.reshape(-1, HIDDEN)

    tk2, w2 = _router(x, W_router)
    tkT = jnp.transpose(tk2).reshape(A)

    sorted_x, ip, blk_exp = _routing_sc(x, tkT)
    y_out = _gmm(
        blk_exp, sorted_x,
        We_gate.astype(jnp.bfloat16),
        We_up.astype(jnp.bfloat16),
        We_down.astype(jnp.bfloat16),
    )
    shared = _shared(
        x,
        Ws_gate.astype(jnp.bfloat16),
        Ws_up.astype(jnp.bfloat16),
        Ws_down.astype(jnp.bfloat16),
    )
    yg = _gather_sc(y_out, ip)
    out = _combine(shared, w2, yg)
    return out.reshape(orig_shape)


# speedup vs baseline: 1.0555x; 1.0555x over previous
import functools

import jax
import jax.numpy as jnp
from jax import lax
from jax.experimental import pallas as pl
from jax.experimental.pallas import tpu as pltpu
from jax.experimental.pallas import tpu_sc as plsc

HIDDEN = 1024
FFN = 2048
MOE_FFN = 512
E = 8
TOP_K = 2
T = 2048
A = T * TOP_K
BLK = 128
NPAD = A + E * BLK
NBLK = NPAD // BLK
TOK_BLK = 256

NW = 16
APT = A // NW


H2 = HIDDEN // 2
_HI_MASK = -65536


def _pack_bf16_pair(lo_f32, hi_f32):
    lo = lax.bitcast_convert_type(
        lo_f32.astype(jnp.bfloat16).astype(jnp.float32), jnp.int32)
    hi = lax.bitcast_convert_type(
        hi_f32.astype(jnp.bfloat16).astype(jnp.float32), jnp.int32)
    return lax.shift_right_logical(lo, 16) | (hi & _HI_MASK)


def _unpack_bf16_pair(words):
    lo = lax.bitcast_convert_type(words << 16, jnp.float32)
    hi = lax.bitcast_convert_type(words & _HI_MASK, jnp.float32)
    return lo, hi


def _router_body(x_ref, wr_ref, tk_ref, w_ref, xp_ref):
    logits = jnp.dot(x_ref[...], wr_ref[...], preferred_element_type=jnp.float32)
    iota = lax.broadcasted_iota(jnp.int32, logits.shape, 1)
    v1 = jnp.max(logits, axis=1, keepdims=True)
    idx1 = jnp.min(jnp.where(logits == v1, iota, E), axis=1, keepdims=True)
    sel1 = iota == idx1
    masked = jnp.where(sel1, -jnp.inf, logits)
    v2 = jnp.max(masked, axis=1, keepdims=True)
    idx2 = jnp.min(jnp.where(masked == v2, iota, E), axis=1, keepdims=True)
    e2 = jnp.exp(v2 - v1)
    w1 = 1.0 / (1.0 + e2)
    w2 = 1.0 - w1
    tk_ref[...] = jnp.concatenate([idx1, idx2], axis=1)
    w_ref[...] = jnp.concatenate([w1, w2], axis=1)
    xp_ref[...] = _pack_bf16_pair(x_ref[:, 0:H2], x_ref[:, H2:HIDDEN])


def _router(x, wr):
    n = T // TOK_BLK
    return pl.pallas_call(
        _router_body,
        grid=(n,),
        in_specs=[
            pl.BlockSpec((TOK_BLK, HIDDEN), lambda t: (t, 0)),
            pl.BlockSpec((HIDDEN, E), lambda t: (0, 0)),
        ],
        out_specs=[
            pl.BlockSpec((TOK_BLK, TOP_K), lambda t: (t, 0)),
            pl.BlockSpec((TOK_BLK, TOP_K), lambda t: (t, 0)),
            pl.BlockSpec((TOK_BLK, H2), lambda t: (t, 0)),
        ],
        out_shape=[
            jax.ShapeDtypeStruct((T, TOP_K), jnp.int32),
            jax.ShapeDtypeStruct((T, TOP_K), jnp.float32),
            jax.ShapeDtypeStruct((T, H2), jnp.int32),
        ],
    )(x, wr)


def _shared_body(x_ref, wg_ref, wu_ref, wd_ref, shared_ref):
    xb = x_ref[...].astype(jnp.bfloat16)
    g = jnp.dot(xb, wg_ref[...], preferred_element_type=jnp.float32)
    u = jnp.dot(xb, wu_ref[...], preferred_element_type=jnp.float32)
    h = (g * jax.nn.sigmoid(g) * u).astype(jnp.bfloat16)
    shared_ref[...] = jnp.dot(h, wd_ref[...], preferred_element_type=jnp.float32)


def _shared(x, wg, wu, wd):
    n = T // TOK_BLK
    return pl.pallas_call(
        _shared_body,
        grid=(n,),
        in_specs=[
            pl.BlockSpec((TOK_BLK, HIDDEN), lambda t: (t, 0)),
            pl.BlockSpec((HIDDEN, FFN), lambda t: (0, 0)),
            pl.BlockSpec((HIDDEN, FFN), lambda t: (0, 0)),
            pl.BlockSpec((FFN, HIDDEN), lambda t: (0, 0)),
        ],
        out_specs=pl.BlockSpec((TOK_BLK, HIDDEN), lambda t: (t, 0)),
        out_shape=jax.ShapeDtypeStruct((T, HIDDEN), jnp.float32),
    )(x, wg, wu, wd)


NW2 = 32
APT2 = A // NW2
VPT = APT2 // 16
NVREG = A // 16


def _routing_sc(x, tkT):
    mesh = plsc.VectorSubcoreMesh(
        core_axis_name="c", subcore_axis_name="s", num_cores=2)

    @functools.partial(
        pl.kernel,
        out_type=[
            jax.ShapeDtypeStruct((NPAD, H2), jnp.int32),
            jax.ShapeDtypeStruct((A,), jnp.int32),
            jax.ShapeDtypeStruct((48,), jnp.int32),
        ],
        mesh=mesh,
        scratch_types=[
            pltpu.VMEM((A,), jnp.int32),
            pltpu.VMEM((2, 64), jnp.int32),
            pltpu.VMEM((64, H2), jnp.int32),
            pltpu.VMEM((16,), jnp.int32),
            pltpu.SemaphoreType.DMA,
        ],
        compiler_params=pltpu.CompilerParams(needs_layout_passes=False),
    )
    def routing(xb_hbm, tkT_hbm, sx_hbm, ip_hbm, be_hbm,
                eids_v, pos_v, rows_v, stage_v, sem):
        wid = lax.axis_index("s") * 2 + lax.axis_index("c")
        iota = lax.iota(jnp.int32, 16)

        pltpu.sync_copy(tkT_hbm, eids_v)

        def scan_body(jj, carry):
            cnt, pre = carry
            pre = jnp.where(jj == wid * VPT, cnt, pre)
            v = eids_v[pl.ds(jj * 16, 16)]
            for e in range(E):
                ce = plsc.all_reduce_population_count(v == e)
                cnt = cnt + jnp.where(iota == e, ce, 0)
            return cnt, pre

        cnt, pre = lax.fori_loop(
            0, NVREG, scan_body,
            (jnp.zeros(16, jnp.int32), jnp.zeros(16, jnp.int32)))
        tot = cnt
        padded = ((tot + (BLK - 1)) >> 7) << 7
        pstart = plsc.cumsum(padded) - padded
        base = pstart + pre

        @pl.when(wid == 0)
        def _():
            sblk = pstart >> 7
            eblk = sblk + (padded >> 7)
            for bj in range(3):
                bvec = bj * 16 + iota
                be = jnp.zeros(16, jnp.int32)
                for e in range(1, E):
                    s_e = jnp.sum(jnp.where(iota == e, sblk, 0))
                    e_e = jnp.sum(jnp.where(iota == e, eblk, 0))
                    be = be + jnp.where((bvec >= s_e) & (bvec < e_e), e, 0)
                stage_v[...] = be
                pltpu.sync_copy(stage_v, be_hbm.at[pl.ds(bj * 16, 16)])

        k_tok0 = lax.rem(wid, NW2 // TOP_K) * APT2
        for q in range(2):
            pos_ref = pos_v.at[q]
            for j in range(4):
                v = eids_v[pl.ds(wid * APT2 + q * 64 + j * 16, 16)]
                pos = jnp.zeros(16, jnp.int32)
                for e in range(E):
                    m = v == e
                    mi = jnp.where(m, 1, 0)
                    excl = plsc.cumsum(mi) - mi
                    b_e = jnp.sum(jnp.where(iota == e, base, 0))
                    pos = pos + mi * (b_e + excl)
                    c_e = plsc.all_reduce_population_count(m)
                    base = base + jnp.where(iota == e, c_e, 0)
                pos_ref[pl.ds(j * 16, 16)] = pos
            pltpu.sync_copy(pos_ref, ip_hbm.at[pl.ds(wid * APT2 + q * 64, 64)])
            pltpu.sync_copy(xb_hbm.at[pl.ds(k_tok0 + q * 64, 64)], rows_v)
            pltpu.async_copy(rows_v, sx_hbm.at[pos_ref], sem).wait()

    return routing(x, tkT)


def _gmm_body(be_ref, xs_ref, wg_ref, wu_ref, wd_ref, y_ref):
    xlo, xhi = _unpack_bf16_pair(xs_ref[...])
    xlo = xlo.astype(jnp.bfloat16)
    xhi = xhi.astype(jnp.bfloat16)
    wg = wg_ref[0]
    wu = wu_ref[0]
    g = (jnp.dot(xlo, wg[0:H2], preferred_element_type=jnp.float32)
         + jnp.dot(xhi, wg[H2:HIDDEN], preferred_element_type=jnp.float32))
    u = (jnp.dot(xlo, wu[0:H2], preferred_element_type=jnp.float32)
         + jnp.dot(xhi, wu[H2:HIDDEN], preferred_element_type=jnp.float32))
    h = (g * jax.nn.sigmoid(g) * u).astype(jnp.bfloat16)
    y = jnp.dot(h, wd_ref[0], preferred_element_type=jnp.float32)
    y_ref[...] = _pack_bf16_pair(y[:, 0:H2], y[:, H2:HIDDEN])


def _gmm(blk_exp, sorted_x, weg, weu, wed):
    grid_spec = pltpu.PrefetchScalarGridSpec(
        num_scalar_prefetch=1,
        grid=(NBLK,),
        in_specs=[
            pl.BlockSpec((BLK, H2), lambda b, be: (b, 0)),
            pl.BlockSpec((1, HIDDEN, MOE_FFN), lambda b, be: (be[b], 0, 0)),
            pl.BlockSpec((1, HIDDEN, MOE_FFN), lambda b, be: (be[b], 0, 0)),
            pl.BlockSpec((1, MOE_FFN, HIDDEN), lambda b, be: (be[b], 0, 0)),
        ],
        out_specs=pl.BlockSpec((BLK, H2), lambda b, be: (b, 0)),
    )
    return pl.pallas_call(
        _gmm_body,
        grid_spec=grid_spec,
        out_shape=jax.ShapeDtypeStruct((NPAD, H2), jnp.int32),
    )(blk_exp, sorted_x, weg, weu, wed)


def _gather_sc(y_out, ip):
    mesh = plsc.VectorSubcoreMesh(
        core_axis_name="c", subcore_axis_name="s", num_cores=2)

    @functools.partial(
        pl.kernel,
        out_type=jax.ShapeDtypeStruct((A, H2), jnp.int32),
        mesh=mesh,
        scratch_types=[
            pltpu.VMEM((2, 64), jnp.int32),
            pltpu.VMEM((64, H2), jnp.int32),
            pltpu.SemaphoreType.DMA,
        ],
        compiler_params=pltpu.CompilerParams(needs_layout_passes=False),
    )
    def gatherk(y_hbm, ip_hbm, yg_hbm, idx_v, gbuf_v, sem):
        wid = lax.axis_index("s") * 2 + lax.axis_index("c")
        for half in range(2):
            base = wid * 128 + half * 64
            idx_ref = idx_v.at[half]
            pltpu.sync_copy(ip_hbm.at[pl.ds(base, 64)], idx_ref)
            pltpu.async_copy(y_hbm.at[idx_ref], gbuf_v, sem).wait()
            pltpu.sync_copy(gbuf_v, yg_hbm.at[pl.ds(base, 64)])

    return gatherk(y_out, ip)


def _combine_body(sh_ref, w_ref, y0_ref, y1_ref, out_ref):
    w = w_ref[...]
    w0 = w[:, 0:1]
    w1 = w[:, 1:2]
    y0lo, y0hi = _unpack_bf16_pair(y0_ref[...])
    y1lo, y1hi = _unpack_bf16_pair(y1_ref[...])
    out_ref[:, 0:H2] = sh_ref[:, 0:H2] + w0 * y0lo + w1 * y1lo
    out_ref[:, H2:HIDDEN] = sh_ref[:, H2:HIDDEN] + w0 * y0hi + w1 * y1hi


def _combine(shared, w2, yg):
    n = T // TOK_BLK
    return pl.pallas_call(
        _combine_body,
        grid=(n,),
        in_specs=[
            pl.BlockSpec((TOK_BLK, HIDDEN), lambda t: (t, 0)),
            pl.BlockSpec((TOK_BLK, TOP_K), lambda t: (t, 0)),
            pl.BlockSpec((TOK_BLK, H2), lambda t: (t, 0)),
            pl.BlockSpec((TOK_BLK, H2), lambda t, n=n: (t + n, 0)),
        ],
        out_specs=pl.BlockSpec((TOK_BLK, HIDDEN), lambda t: (t, 0)),
        out_shape=jax.ShapeDtypeStruct((T, HIDDEN), jnp.float32),
    )(shared, w2, yg, yg)


def kernel(hidden_states, W_router, Ws_gate, Ws_up, Ws_down,
           We_gate, We_up, We_down):
    orig_shape = hidden_states.shape
    x = hidden_states.reshape(-1, HIDDEN)

    tk2, w2, xpack = _router(x, W_router)
    tkT = jnp.transpose(tk2).reshape(A)

    sorted_x, ip, blk_exp = _routing_sc(xpack, tkT)
    y_out = _gmm(
        blk_exp, sorted_x,
        We_gate.astype(jnp.bfloat16),
        We_up.astype(jnp.bfloat16),
        We_down.astype(jnp.bfloat16),
    )
    shared = _shared(
        x,
        Ws_gate.astype(jnp.bfloat16),
        Ws_up.astype(jnp.bfloat16),
        Ws_down.astype(jnp.bfloat16),
    )
    yg = _gather_sc(y_out, ip)
    out = _combine(shared, w2, yg)
    return out.reshape(orig_shape)


# speedup vs baseline: 1.2234x; 1.1590x over previous
import functools

import jax
import jax.numpy as jnp
from jax import lax
from jax.experimental import pallas as pl
from jax.experimental.pallas import tpu as pltpu
from jax.experimental.pallas import tpu_sc as plsc

HIDDEN = 1024
FFN = 2048
MOE_FFN = 512
E = 8
TOP_K = 2
T = 2048
A = T * TOP_K
BLK = 128
NPAD = A + E * BLK
NBLK = NPAD // BLK
TOK_BLK = 256

NW = 16
APT = A // NW


H2 = HIDDEN // 2
_HI_MASK = -65536


def _pack_bf16_pair(lo_f32, hi_f32):
    lo = lax.bitcast_convert_type(
        lo_f32.astype(jnp.bfloat16).astype(jnp.float32), jnp.int32)
    hi = lax.bitcast_convert_type(
        hi_f32.astype(jnp.bfloat16).astype(jnp.float32), jnp.int32)
    return lax.shift_right_logical(lo, 16) | (hi & _HI_MASK)


def _unpack_bf16_pair(words):
    lo = lax.bitcast_convert_type(words << 16, jnp.float32)
    hi = lax.bitcast_convert_type(words & _HI_MASK, jnp.float32)
    return lo, hi


def _router_body(x_ref, wr_ref, tk_ref, w_ref, xp_ref):
    logits = jnp.dot(x_ref[...], wr_ref[...], preferred_element_type=jnp.float32)
    iota = lax.broadcasted_iota(jnp.int32, logits.shape, 1)
    v1 = jnp.max(logits, axis=1, keepdims=True)
    idx1 = jnp.min(jnp.where(logits == v1, iota, E), axis=1, keepdims=True)
    sel1 = iota == idx1
    masked = jnp.where(sel1, -jnp.inf, logits)
    v2 = jnp.max(masked, axis=1, keepdims=True)
    idx2 = jnp.min(jnp.where(masked == v2, iota, E), axis=1, keepdims=True)
    e2 = jnp.exp(v2 - v1)
    w1 = 1.0 / (1.0 + e2)
    w2 = 1.0 - w1
    tk_ref[...] = jnp.concatenate([idx1, idx2], axis=1)
    w_ref[...] = jnp.concatenate([w1, w2], axis=1)
    xp_ref[...] = _pack_bf16_pair(x_ref[:, 0:H2], x_ref[:, H2:HIDDEN])


def _router(x, wr):
    n = T // TOK_BLK
    return pl.pallas_call(
        _router_body,
        grid=(n,),
        in_specs=[
            pl.BlockSpec((TOK_BLK, HIDDEN), lambda t: (t, 0)),
            pl.BlockSpec((HIDDEN, E), lambda t: (0, 0)),
        ],
        out_specs=[
            pl.BlockSpec((TOK_BLK, TOP_K), lambda t: (t, 0)),
            pl.BlockSpec((TOK_BLK, TOP_K), lambda t: (t, 0)),
            pl.BlockSpec((TOK_BLK, H2), lambda t: (t, 0)),
        ],
        out_shape=[
            jax.ShapeDtypeStruct((T, TOP_K), jnp.int32),
            jax.ShapeDtypeStruct((T, TOP_K), jnp.float32),
            jax.ShapeDtypeStruct((T, H2), jnp.int32),
        ],
    )(x, wr)


def _shared_body(x_ref, wg_ref, wu_ref, wd_ref, shared_ref):
    x = x_ref[...]
    g = jnp.dot(x, wg_ref[...], preferred_element_type=jnp.float32)
    u = jnp.dot(x, wu_ref[...], preferred_element_type=jnp.float32)
    h = g * jax.nn.sigmoid(g) * u
    shared_ref[...] = jnp.dot(h, wd_ref[...], preferred_element_type=jnp.float32)


def _shared(x, wg, wu, wd):
    n = T // TOK_BLK
    return pl.pallas_call(
        _shared_body,
        grid=(n,),
        in_specs=[
            pl.BlockSpec((TOK_BLK, HIDDEN), lambda t: (t, 0)),
            pl.BlockSpec((HIDDEN, FFN), lambda t: (0, 0)),
            pl.BlockSpec((HIDDEN, FFN), lambda t: (0, 0)),
            pl.BlockSpec((FFN, HIDDEN), lambda t: (0, 0)),
        ],
        out_specs=pl.BlockSpec((TOK_BLK, HIDDEN), lambda t: (t, 0)),
        out_shape=jax.ShapeDtypeStruct((T, HIDDEN), jnp.float32),
    )(x, wg, wu, wd)


NW2 = 32
APT2 = A // NW2
VPT = APT2 // 16
NVREG = A // 16


def _routing_sc(x, tkT):
    mesh = plsc.VectorSubcoreMesh(
        core_axis_name="c", subcore_axis_name="s", num_cores=2)

    @functools.partial(
        pl.kernel,
        out_type=[
            jax.ShapeDtypeStruct((NPAD, H2), jnp.int32),
            jax.ShapeDtypeStruct((A,), jnp.int32),
            jax.ShapeDtypeStruct((48,), jnp.int32),
        ],
        mesh=mesh,
        scratch_types=[
            pltpu.VMEM((A,), jnp.int32),
            pltpu.VMEM((2, 64), jnp.int32),
            pltpu.VMEM((64, H2), jnp.int32),
            pltpu.VMEM((16,), jnp.int32),
            pltpu.SemaphoreType.DMA,
        ],
        compiler_params=pltpu.CompilerParams(needs_layout_passes=False),
    )
    def routing(xb_hbm, tkT_hbm, sx_hbm, ip_hbm, be_hbm,
                eids_v, pos_v, rows_v, stage_v, sem):
        wid = lax.axis_index("s") * 2 + lax.axis_index("c")
        iota = lax.iota(jnp.int32, 16)

        pltpu.sync_copy(tkT_hbm, eids_v)

        def scan_body(jj, carry):
            cnt, pre = carry
            pre = jnp.where(jj == wid * VPT, cnt, pre)
            v = eids_v[pl.ds(jj * 16, 16)]
            for e in range(E):
                ce = plsc.all_reduce_population_count(v == e)
                cnt = cnt + jnp.where(iota == e, ce, 0)
            return cnt, pre

        cnt, pre = lax.fori_loop(
            0, NVREG, scan_body,
            (jnp.zeros(16, jnp.int32), jnp.zeros(16, jnp.int32)))
        tot = cnt
        padded = ((tot + (BLK - 1)) >> 7) << 7
        pstart = plsc.cumsum(padded) - padded
        base = pstart + pre

        @pl.when(wid == 0)
        def _():
            sblk = pstart >> 7
            eblk = sblk + (padded >> 7)
            for bj in range(3):
                bvec = bj * 16 + iota
                be = jnp.zeros(16, jnp.int32)
                for e in range(1, E):
                    s_e = jnp.sum(jnp.where(iota == e, sblk, 0))
                    e_e = jnp.sum(jnp.where(iota == e, eblk, 0))
                    be = be + jnp.where((bvec >= s_e) & (bvec < e_e), e, 0)
                stage_v[...] = be
                pltpu.sync_copy(stage_v, be_hbm.at[pl.ds(bj * 16, 16)])

        k_tok0 = lax.rem(wid, NW2 // TOP_K) * APT2
        for q in range(2):
            pos_ref = pos_v.at[q]
            for j in range(4):
                v = eids_v[pl.ds(wid * APT2 + q * 64 + j * 16, 16)]
                pos = jnp.zeros(16, jnp.int32)
                for e in range(E):
                    m = v == e
                    mi = jnp.where(m, 1, 0)
                    excl = plsc.cumsum(mi) - mi
                    b_e = jnp.sum(jnp.where(iota == e, base, 0))
                    pos = pos + mi * (b_e + excl)
                    c_e = plsc.all_reduce_population_count(m)
                    base = base + jnp.where(iota == e, c_e, 0)
                pos_ref[pl.ds(j * 16, 16)] = pos
            pltpu.sync_copy(pos_ref, ip_hbm.at[pl.ds(wid * APT2 + q * 64, 64)])
            pltpu.sync_copy(xb_hbm.at[pl.ds(k_tok0 + q * 64, 64)], rows_v)
            pltpu.async_copy(rows_v, sx_hbm.at[pos_ref], sem).wait()

    return routing(x, tkT)


def _gmm_body(be_ref, xs_ref, wg_ref, wu_ref, wd_ref, y_ref):
    xlo, xhi = _unpack_bf16_pair(xs_ref[...])
    wg = wg_ref[0]
    wu = wu_ref[0]
    g = (jnp.dot(xlo, wg[0:H2], preferred_element_type=jnp.float32)
         + jnp.dot(xhi, wg[H2:HIDDEN], preferred_element_type=jnp.float32))
    u = (jnp.dot(xlo, wu[0:H2], preferred_element_type=jnp.float32)
         + jnp.dot(xhi, wu[H2:HIDDEN], preferred_element_type=jnp.float32))
    h = g * jax.nn.sigmoid(g) * u
    y = jnp.dot(h, wd_ref[0], preferred_element_type=jnp.float32)
    y_ref[...] = _pack_bf16_pair(y[:, 0:H2], y[:, H2:HIDDEN])


def _gmm(blk_exp, sorted_x, weg, weu, wed):
    grid_spec = pltpu.PrefetchScalarGridSpec(
        num_scalar_prefetch=1,
        grid=(NBLK,),
        in_specs=[
            pl.BlockSpec((BLK, H2), lambda b, be: (b, 0)),
            pl.BlockSpec((1, HIDDEN, MOE_FFN), lambda b, be: (be[b], 0, 0)),
            pl.BlockSpec((1, HIDDEN, MOE_FFN), lambda b, be: (be[b], 0, 0)),
            pl.BlockSpec((1, MOE_FFN, HIDDEN), lambda b, be: (be[b], 0, 0)),
        ],
        out_specs=pl.BlockSpec((BLK, H2), lambda b, be: (b, 0)),
    )
    return pl.pallas_call(
        _gmm_body,
        grid_spec=grid_spec,
        out_shape=jax.ShapeDtypeStruct((NPAD, H2), jnp.int32),
    )(blk_exp, sorted_x, weg, weu, wed)


def _gather_sc(y_out, ip):
    mesh = plsc.VectorSubcoreMesh(
        core_axis_name="c", subcore_axis_name="s", num_cores=2)

    @functools.partial(
        pl.kernel,
        out_type=jax.ShapeDtypeStruct((A, H2), jnp.int32),
        mesh=mesh,
        scratch_types=[
            pltpu.VMEM((2, 64), jnp.int32),
            pltpu.VMEM((64, H2), jnp.int32),
            pltpu.SemaphoreType.DMA,
        ],
        compiler_params=pltpu.CompilerParams(needs_layout_passes=False),
    )
    def gatherk(y_hbm, ip_hbm, yg_hbm, idx_v, gbuf_v, sem):
        wid = lax.axis_index("s") * 2 + lax.axis_index("c")
        for half in range(2):
            base = wid * 128 + half * 64
            idx_ref = idx_v.at[half]
            pltpu.sync_copy(ip_hbm.at[pl.ds(base, 64)], idx_ref)
            pltpu.async_copy(y_hbm.at[idx_ref], gbuf_v, sem).wait()
            pltpu.sync_copy(gbuf_v, yg_hbm.at[pl.ds(base, 64)])

    return gatherk(y_out, ip)


def _combine_body(sh_ref, w_ref, y0_ref, y1_ref, out_ref):
    w = w_ref[...]
    w0 = w[:, 0:1]
    w1 = w[:, 1:2]
    y0lo, y0hi = _unpack_bf16_pair(y0_ref[...])
    y1lo, y1hi = _unpack_bf16_pair(y1_ref[...])
    out_ref[:, 0:H2] = sh_ref[:, 0:H2] + w0 * y0lo + w1 * y1lo
    out_ref[:, H2:HIDDEN] = sh_ref[:, H2:HIDDEN] + w0 * y0hi + w1 * y1hi


def _combine(shared, w2, yg):
    n = T // TOK_BLK
    return pl.pallas_call(
        _combine_body,
        grid=(n,),
        in_specs=[
            pl.BlockSpec((TOK_BLK, HIDDEN), lambda t: (t, 0)),
            pl.BlockSpec((TOK_BLK, TOP_K), lambda t: (t, 0)),
            pl.BlockSpec((TOK_BLK, H2), lambda t: (t, 0)),
            pl.BlockSpec((TOK_BLK, H2), lambda t, n=n: (t + n, 0)),
        ],
        out_specs=pl.BlockSpec((TOK_BLK, HIDDEN), lambda t: (t, 0)),
        out_shape=jax.ShapeDtypeStruct((T, HIDDEN), jnp.float32),
    )(shared, w2, yg, yg)


def kernel(hidden_states, W_router, Ws_gate, Ws_up, Ws_down,
           We_gate, We_up, We_down):
    orig_shape = hidden_states.shape
    x = hidden_states.reshape(-1, HIDDEN)

    tk2, w2, xpack = _router(x, W_router)
    tkT = jnp.transpose(tk2).reshape(A)

    sorted_x, ip, blk_exp = _routing_sc(xpack, tkT)
    y_out = _gmm(blk_exp, sorted_x, We_gate, We_up, We_down)
    shared = _shared(x, Ws_gate, Ws_up, Ws_down)
    yg = _gather_sc(y_out, ip)
    out = _combine(shared, w2, yg)
    return out.reshape(orig_shape)


# speedup vs baseline: 1.3114x; 1.0719x over previous
import functools

import jax
import jax.numpy as jnp
from jax import lax
from jax.experimental import pallas as pl
from jax.experimental.pallas import tpu as pltpu
from jax.experimental.pallas import tpu_sc as plsc

HIDDEN = 1024
FFN = 2048
MOE_FFN = 512
E = 8
TOP_K = 2
T = 2048
A = T * TOP_K
BLK = 256
BLK_SHIFT = 8
NPAD = A + E * BLK
NBLK = NPAD // BLK
TOK_BLK = 256

NW = 16
APT = A // NW


H2 = HIDDEN // 2
_HI_MASK = -65536


def _pack_bf16_pair(lo_f32, hi_f32):
    lo = lax.bitcast_convert_type(
        lo_f32.astype(jnp.bfloat16).astype(jnp.float32), jnp.int32)
    hi = lax.bitcast_convert_type(
        hi_f32.astype(jnp.bfloat16).astype(jnp.float32), jnp.int32)
    return lax.shift_right_logical(lo, 16) | (hi & _HI_MASK)


def _unpack_bf16_pair(words):
    lo = lax.bitcast_convert_type(words << 16, jnp.float32)
    hi = lax.bitcast_convert_type(words & _HI_MASK, jnp.float32)
    return lo, hi


def _router_body(x_ref, wr_ref, tk_ref, w_ref, xp_ref):
    logits = jnp.dot(x_ref[...], wr_ref[...], preferred_element_type=jnp.float32)
    iota = lax.broadcasted_iota(jnp.int32, logits.shape, 1)
    v1 = jnp.max(logits, axis=1, keepdims=True)
    idx1 = jnp.min(jnp.where(logits == v1, iota, E), axis=1, keepdims=True)
    sel1 = iota == idx1
    masked = jnp.where(sel1, -jnp.inf, logits)
    v2 = jnp.max(masked, axis=1, keepdims=True)
    idx2 = jnp.min(jnp.where(masked == v2, iota, E), axis=1, keepdims=True)
    e2 = jnp.exp(v2 - v1)
    w1 = 1.0 / (1.0 + e2)
    w2 = 1.0 - w1
    tk_ref[...] = jnp.concatenate([idx1, idx2], axis=1)
    w_ref[...] = jnp.concatenate([w1, w2], axis=1)
    xp_ref[...] = _pack_bf16_pair(x_ref[:, 0:H2], x_ref[:, H2:HIDDEN])


def _router(x, wr):
    n = T // TOK_BLK
    return pl.pallas_call(
        _router_body,
        grid=(n,),
        in_specs=[
            pl.BlockSpec((TOK_BLK, HIDDEN), lambda t: (t, 0)),
            pl.BlockSpec((HIDDEN, E), lambda t: (0, 0)),
        ],
        out_specs=[
            pl.BlockSpec((TOK_BLK, TOP_K), lambda t: (t, 0)),
            pl.BlockSpec((TOK_BLK, TOP_K), lambda t: (t, 0)),
            pl.BlockSpec((TOK_BLK, H2), lambda t: (t, 0)),
        ],
        out_shape=[
            jax.ShapeDtypeStruct((T, TOP_K), jnp.int32),
            jax.ShapeDtypeStruct((T, TOP_K), jnp.float32),
            jax.ShapeDtypeStruct((T, H2), jnp.int32),
        ],
    )(x, wr)


def _shared_combine_body(x_ref, wg_ref, wu_ref, wd_ref, w_ref, y0_ref, y1_ref,
                         out_ref):
    x = x_ref[...]
    g = jnp.dot(x, wg_ref[...], preferred_element_type=jnp.float32)
    u = jnp.dot(x, wu_ref[...], preferred_element_type=jnp.float32)
    h = g * jax.nn.sigmoid(g) * u
    sh = jnp.dot(h, wd_ref[...], preferred_element_type=jnp.float32)
    w = w_ref[...]
    w0 = w[:, 0:1]
    w1 = w[:, 1:2]
    y0lo, y0hi = _unpack_bf16_pair(y0_ref[...])
    y1lo, y1hi = _unpack_bf16_pair(y1_ref[...])
    out_ref[:, 0:H2] = sh[:, 0:H2] + w0 * y0lo + w1 * y1lo
    out_ref[:, H2:HIDDEN] = sh[:, H2:HIDDEN] + w0 * y0hi + w1 * y1hi


def _shared_combine(x, wg, wu, wd, w2, yg):
    n = T // TOK_BLK
    return pl.pallas_call(
        _shared_combine_body,
        grid=(n,),
        in_specs=[
            pl.BlockSpec((TOK_BLK, HIDDEN), lambda t: (t, 0)),
            pl.BlockSpec((HIDDEN, FFN), lambda t: (0, 0)),
            pl.BlockSpec((HIDDEN, FFN), lambda t: (0, 0)),
            pl.BlockSpec((FFN, HIDDEN), lambda t: (0, 0)),
            pl.BlockSpec((TOK_BLK, TOP_K), lambda t: (t, 0)),
            pl.BlockSpec((TOK_BLK, H2), lambda t: (t, 0)),
            pl.BlockSpec((TOK_BLK, H2), lambda t, n=n: (t + n, 0)),
        ],
        out_specs=pl.BlockSpec((TOK_BLK, HIDDEN), lambda t: (t, 0)),
        out_shape=jax.ShapeDtypeStruct((T, HIDDEN), jnp.float32),
    )(x, wg, wu, wd, w2, yg, yg)


NW2 = 32
APT2 = A // NW2
VPT = APT2 // 16
NVREG = A // 16


def _routing_sc(x, tkT):
    mesh = plsc.VectorSubcoreMesh(
        core_axis_name="c", subcore_axis_name="s", num_cores=2)

    @functools.partial(
        pl.kernel,
        out_type=[
            jax.ShapeDtypeStruct((NPAD, H2), jnp.int32),
            jax.ShapeDtypeStruct((A,), jnp.int32),
            jax.ShapeDtypeStruct((48,), jnp.int32),
        ],
        mesh=mesh,
        scratch_types=[
            pltpu.VMEM((A,), jnp.int32),
            pltpu.VMEM((2, 64), jnp.int32),
            pltpu.VMEM((64, H2), jnp.int32),
            pltpu.VMEM((16,), jnp.int32),
            pltpu.SemaphoreType.DMA,
        ],
        compiler_params=pltpu.CompilerParams(needs_layout_passes=False),
    )
    def routing(xb_hbm, tkT_hbm, sx_hbm, ip_hbm, be_hbm,
                eids_v, pos_v, rows_v, stage_v, sem):
        wid = lax.axis_index("s") * 2 + lax.axis_index("c")
        iota = lax.iota(jnp.int32, 16)

        pltpu.sync_copy(tkT_hbm, eids_v)

        def scan_body(jj, carry):
            cnt, pre = carry
            pre = jnp.where(jj == wid * VPT, cnt, pre)
            v = eids_v[pl.ds(jj * 16, 16)]
            for e in range(E):
                ce = plsc.all_reduce_population_count(v == e)
                cnt = cnt + jnp.where(iota == e, ce, 0)
            return cnt, pre

        cnt, pre = lax.fori_loop(
            0, NVREG, scan_body,
            (jnp.zeros(16, jnp.int32), jnp.zeros(16, jnp.int32)))
        tot = cnt
        padded = ((tot + (BLK - 1)) >> BLK_SHIFT) << BLK_SHIFT
        pstart = plsc.cumsum(padded) - padded
        base = pstart + pre

        @pl.when(wid == 0)
        def _():
            sblk = pstart >> BLK_SHIFT
            eblk = sblk + (padded >> BLK_SHIFT)
            for bj in range(3):
                bvec = bj * 16 + iota
                be = jnp.zeros(16, jnp.int32)
                for e in range(1, E):
                    s_e = jnp.sum(jnp.where(iota == e, sblk, 0))
                    e_e = jnp.sum(jnp.where(iota == e, eblk, 0))
                    be = be + jnp.where((bvec >= s_e) & (bvec < e_e), e, 0)
                stage_v[...] = be
                pltpu.sync_copy(stage_v, be_hbm.at[pl.ds(bj * 16, 16)])

        k_tok0 = lax.rem(wid, NW2 // TOP_K) * APT2
        for q in range(2):
            pos_ref = pos_v.at[q]
            for j in range(4):
                v = eids_v[pl.ds(wid * APT2 + q * 64 + j * 16, 16)]
                pos = jnp.zeros(16, jnp.int32)
                for e in range(E):
                    m = v == e
                    mi = jnp.where(m, 1, 0)
                    excl = plsc.cumsum(mi) - mi
                    b_e = jnp.sum(jnp.where(iota == e, base, 0))
                    pos = pos + mi * (b_e + excl)
                    c_e = plsc.all_reduce_population_count(m)
                    base = base + jnp.where(iota == e, c_e, 0)
                pos_ref[pl.ds(j * 16, 16)] = pos
            pltpu.sync_copy(pos_ref, ip_hbm.at[pl.ds(wid * APT2 + q * 64, 64)])
            pltpu.sync_copy(xb_hbm.at[pl.ds(k_tok0 + q * 64, 64)], rows_v)
            pltpu.async_copy(rows_v, sx_hbm.at[pos_ref], sem).wait()

    return routing(x, tkT)


def _gmm_body(be_ref, xs_ref, wg_ref, wu_ref, wd_ref, y_ref):
    xlo, xhi = _unpack_bf16_pair(xs_ref[...])
    wg = wg_ref[0]
    wu = wu_ref[0]
    g = (jnp.dot(xlo, wg[0:H2], preferred_element_type=jnp.float32)
         + jnp.dot(xhi, wg[H2:HIDDEN], preferred_element_type=jnp.float32))
    u = (jnp.dot(xlo, wu[0:H2], preferred_element_type=jnp.float32)
         + jnp.dot(xhi, wu[H2:HIDDEN], preferred_element_type=jnp.float32))
    h = g * jax.nn.sigmoid(g) * u
    y = jnp.dot(h, wd_ref[0], preferred_element_type=jnp.float32)
    y_ref[...] = _pack_bf16_pair(y[:, 0:H2], y[:, H2:HIDDEN])


def _gmm(blk_exp, sorted_x, weg, weu, wed):
    grid_spec = pltpu.PrefetchScalarGridSpec(
        num_scalar_prefetch=1,
        grid=(NBLK,),
        in_specs=[
            pl.BlockSpec((BLK, H2), lambda b, be: (b, 0)),
            pl.BlockSpec((1, HIDDEN, MOE_FFN), lambda b, be: (be[b], 0, 0)),
            pl.BlockSpec((1, HIDDEN, MOE_FFN), lambda b, be: (be[b], 0, 0)),
            pl.BlockSpec((1, MOE_FFN, HIDDEN), lambda b, be: (be[b], 0, 0)),
        ],
        out_specs=pl.BlockSpec((BLK, H2), lambda b, be: (b, 0)),
    )
    return pl.pallas_call(
        _gmm_body,
        grid_spec=grid_spec,
        out_shape=jax.ShapeDtypeStruct((NPAD, H2), jnp.int32),
    )(blk_exp, sorted_x, weg, weu, wed)


def _gather_sc(y_out, ip):
    mesh = plsc.VectorSubcoreMesh(
        core_axis_name="c", subcore_axis_name="s", num_cores=2)

    @functools.partial(
        pl.kernel,
        out_type=jax.ShapeDtypeStruct((A, H2), jnp.int32),
        mesh=mesh,
        scratch_types=[
            pltpu.VMEM((2, 64), jnp.int32),
            pltpu.VMEM((64, H2), jnp.int32),
            pltpu.SemaphoreType.DMA,
        ],
        compiler_params=pltpu.CompilerParams(needs_layout_passes=False),
    )
    def gatherk(y_hbm, ip_hbm, yg_hbm, idx_v, gbuf_v, sem):
        wid = lax.axis_index("s") * 2 + lax.axis_index("c")
        for half in range(2):
            base = wid * 128 + half * 64
            idx_ref = idx_v.at[half]
            pltpu.sync_copy(ip_hbm.at[pl.ds(base, 64)], idx_ref)
            pltpu.async_copy(y_hbm.at[idx_ref], gbuf_v, sem).wait()
            pltpu.sync_copy(gbuf_v, yg_hbm.at[pl.ds(base, 64)])

    return gatherk(y_out, ip)


def _combine_body(sh_ref, w_ref, y0_ref, y1_ref, out_ref):
    w = w_ref[...]
    w0 = w[:, 0:1]
    w1 = w[:, 1:2]
    y0lo, y0hi = _unpack_bf16_pair(y0_ref[...])
    y1lo, y1hi = _unpack_bf16_pair(y1_ref[...])
    out_ref[:, 0:H2] = sh_ref[:, 0:H2] + w0 * y0lo + w1 * y1lo
    out_ref[:, H2:HIDDEN] = sh_ref[:, H2:HIDDEN] + w0 * y0hi + w1 * y1hi


def _combine(shared, w2, yg):
    n = T // TOK_BLK
    return pl.pallas_call(
        _combine_body,
        grid=(n,),
        in_specs=[
            pl.BlockSpec((TOK_BLK, HIDDEN), lambda t: (t, 0)),
            pl.BlockSpec((TOK_BLK, TOP_K), lambda t: (t, 0)),
            pl.BlockSpec((TOK_BLK, H2), lambda t: (t, 0)),
            pl.BlockSpec((TOK_BLK, H2), lambda t, n=n: (t + n, 0)),
        ],
        out_specs=pl.BlockSpec((TOK_BLK, HIDDEN), lambda t: (t, 0)),
        out_shape=jax.ShapeDtypeStruct((T, HIDDEN), jnp.float32),
    )(shared, w2, yg, yg)


def kernel(hidden_states, W_router, Ws_gate, Ws_up, Ws_down,
           We_gate, We_up, We_down):
    orig_shape = hidden_states.shape
    x = hidden_states.reshape(-1, HIDDEN)

    tk2, w2, xpack = _router(x, W_router)
    tkT = jnp.transpose(tk2).reshape(A)

    sorted_x, ip, blk_exp = _routing_sc(xpack, tkT)
    y_out = _gmm(blk_exp, sorted_x, We_gate, We_up, We_down)
    yg = _gather_sc(y_out, ip)
    out = _shared_combine(x, Ws_gate, Ws_up, Ws_down, w2, yg)
    return out.reshape(orig_shape)


# speedup vs baseline: 1.3599x; 1.0371x over previous
import functools

import jax
import jax.numpy as jnp
from jax import lax
from jax.experimental import pallas as pl
from jax.experimental.pallas import tpu as pltpu
from jax.experimental.pallas import tpu_sc as plsc

HIDDEN = 1024
FFN = 2048
MOE_FFN = 512
E = 8
TOP_K = 2
T = 2048
A = T * TOP_K
BLK = 256
BLK_SHIFT = 8
NPAD = A + E * BLK
NBLK = NPAD // BLK
TOK_BLK = 256

NW = 16
APT = A // NW


H2 = HIDDEN // 2
_HI_MASK = -65536


def _pack_bf16_pair(lo_f32, hi_f32):
    lo = lax.bitcast_convert_type(
        lo_f32.astype(jnp.bfloat16).astype(jnp.float32), jnp.int32)
    hi = lax.bitcast_convert_type(
        hi_f32.astype(jnp.bfloat16).astype(jnp.float32), jnp.int32)
    return lax.shift_right_logical(lo, 16) | (hi & _HI_MASK)


def _unpack_bf16_pair(words):
    lo = lax.bitcast_convert_type(words << 16, jnp.float32)
    hi = lax.bitcast_convert_type(words & _HI_MASK, jnp.float32)
    return lo, hi


def _router_body(x_ref, wr_ref, tk_ref, w_ref, xp_ref):
    logits = jnp.dot(x_ref[...], wr_ref[...], preferred_element_type=jnp.float32)
    iota = lax.broadcasted_iota(jnp.int32, logits.shape, 1)
    v1 = jnp.max(logits, axis=1, keepdims=True)
    idx1 = jnp.min(jnp.where(logits == v1, iota, E), axis=1, keepdims=True)
    sel1 = iota == idx1
    masked = jnp.where(sel1, -jnp.inf, logits)
    v2 = jnp.max(masked, axis=1, keepdims=True)
    idx2 = jnp.min(jnp.where(masked == v2, iota, E), axis=1, keepdims=True)
    e2 = jnp.exp(v2 - v1)
    w1 = 1.0 / (1.0 + e2)
    w2 = 1.0 - w1
    tk_ref[...] = jnp.concatenate([idx1, idx2], axis=1)
    w_ref[...] = jnp.concatenate([w1, w2], axis=1)
    xp_ref[...] = _pack_bf16_pair(x_ref[:, 0:H2], x_ref[:, H2:HIDDEN])


def _router(x, wr):
    n = T // TOK_BLK
    return pl.pallas_call(
        _router_body,
        grid=(n,),
        in_specs=[
            pl.BlockSpec((TOK_BLK, HIDDEN), lambda t: (t, 0)),
            pl.BlockSpec((HIDDEN, E), lambda t: (0, 0)),
        ],
        out_specs=[
            pl.BlockSpec((TOK_BLK, TOP_K), lambda t: (t, 0)),
            pl.BlockSpec((TOK_BLK, TOP_K), lambda t: (t, 0)),
            pl.BlockSpec((TOK_BLK, H2), lambda t: (t, 0)),
        ],
        out_shape=[
            jax.ShapeDtypeStruct((T, TOP_K), jnp.int32),
            jax.ShapeDtypeStruct((T, TOP_K), jnp.float32),
            jax.ShapeDtypeStruct((T, H2), jnp.int32),
        ],
    )(x, wr)


def _shared_combine_body(x_ref, wg_ref, wu_ref, wd_ref, w_ref, y0_ref, y1_ref,
                         out_ref):
    x = x_ref[...]
    g = jnp.dot(x, wg_ref[...], preferred_element_type=jnp.float32)
    u = jnp.dot(x, wu_ref[...], preferred_element_type=jnp.float32)
    h = g * jax.nn.sigmoid(g) * u
    sh = jnp.dot(h, wd_ref[...], preferred_element_type=jnp.float32)
    w = w_ref[...]
    w0 = w[:, 0:1]
    w1 = w[:, 1:2]
    y0lo, y0hi = _unpack_bf16_pair(y0_ref[...])
    y1lo, y1hi = _unpack_bf16_pair(y1_ref[...])
    out_ref[:, 0:H2] = sh[:, 0:H2] + w0 * y0lo + w1 * y1lo
    out_ref[:, H2:HIDDEN] = sh[:, H2:HIDDEN] + w0 * y0hi + w1 * y1hi


def _shared_combine(x, wg, wu, wd, w2, yg):
    n = T // TOK_BLK
    return pl.pallas_call(
        _shared_combine_body,
        grid=(n,),
        in_specs=[
            pl.BlockSpec((TOK_BLK, HIDDEN), lambda t: (t, 0)),
            pl.BlockSpec((HIDDEN, FFN), lambda t: (0, 0)),
            pl.BlockSpec((HIDDEN, FFN), lambda t: (0, 0)),
            pl.BlockSpec((FFN, HIDDEN), lambda t: (0, 0)),
            pl.BlockSpec((TOK_BLK, TOP_K), lambda t: (t, 0)),
            pl.BlockSpec((TOK_BLK, H2), lambda t: (t, 0)),
            pl.BlockSpec((TOK_BLK, H2), lambda t, n=n: (t + n, 0)),
        ],
        out_specs=pl.BlockSpec((TOK_BLK, HIDDEN), lambda t: (t, 0)),
        out_shape=jax.ShapeDtypeStruct((T, HIDDEN), jnp.float32),
    )(x, wg, wu, wd, w2, yg, yg)


NW2 = 32
APT2 = A // NW2
VPT = APT2 // 16
NVREG = A // 16


def _routing_sc(x, tkT):
    mesh = plsc.VectorSubcoreMesh(
        core_axis_name="c", subcore_axis_name="s", num_cores=2)

    @functools.partial(
        pl.kernel,
        out_type=[
            jax.ShapeDtypeStruct((NPAD, H2), jnp.int32),
            jax.ShapeDtypeStruct((A,), jnp.int32),
            jax.ShapeDtypeStruct((48,), jnp.int32),
        ],
        mesh=mesh,
        scratch_types=[
            pltpu.VMEM((A,), jnp.int32),
            pltpu.VMEM((2, 64), jnp.int32),
            pltpu.VMEM((64, H2), jnp.int32),
            pltpu.VMEM((64, H2), jnp.int32),
            pltpu.VMEM((16,), jnp.int32),
            pltpu.SemaphoreType.DMA,
            pltpu.SemaphoreType.DMA,
        ],
        compiler_params=pltpu.CompilerParams(needs_layout_passes=False),
    )
    def routing(xb_hbm, tkT_hbm, sx_hbm, ip_hbm, be_hbm,
                eids_v, pos_v, rows0_v, rows1_v, stage_v, sem, sem2):
        wid = lax.axis_index("s") * 2 + lax.axis_index("c")
        iota = lax.iota(jnp.int32, 16)

        k_tok0 = lax.rem(wid, NW2 // TOP_K) * APT2
        ld0 = pltpu.async_copy(xb_hbm.at[pl.ds(k_tok0, 64)], rows0_v, sem)
        ld1 = pltpu.async_copy(xb_hbm.at[pl.ds(k_tok0 + 64, 64)], rows1_v, sem2)

        pltpu.sync_copy(tkT_hbm, eids_v)

        def scan_body(jj, carry):
            cnt, pre = carry
            pre = jnp.where(jj == wid * VPT, cnt, pre)
            v = eids_v[pl.ds(jj * 16, 16)]
            for e in range(E):
                ce = plsc.all_reduce_population_count(v == e)
                cnt = cnt + jnp.where(iota == e, ce, 0)
            return cnt, pre

        cnt, pre = lax.fori_loop(
            0, NVREG, scan_body,
            (jnp.zeros(16, jnp.int32), jnp.zeros(16, jnp.int32)))
        tot = cnt
        padded = ((tot + (BLK - 1)) >> BLK_SHIFT) << BLK_SHIFT
        pstart = plsc.cumsum(padded) - padded
        base = pstart + pre

        @pl.when(wid == 0)
        def _():
            sblk = pstart >> BLK_SHIFT
            eblk = sblk + (padded >> BLK_SHIFT)
            for bj in range(3):
                bvec = bj * 16 + iota
                be = jnp.zeros(16, jnp.int32)
                for e in range(1, E):
                    s_e = jnp.sum(jnp.where(iota == e, sblk, 0))
                    e_e = jnp.sum(jnp.where(iota == e, eblk, 0))
                    be = be + jnp.where((bvec >= s_e) & (bvec < e_e), e, 0)
                stage_v[...] = be
                pltpu.sync_copy(stage_v, be_hbm.at[pl.ds(bj * 16, 16)])

        rows = [rows0_v, rows1_v]
        lds = [ld0, ld1]
        sems = [sem, sem2]
        scat = [None, None]
        for q in range(2):
            pos_ref = pos_v.at[q]
            for j in range(4):
                v = eids_v[pl.ds(wid * APT2 + q * 64 + j * 16, 16)]
                pos = jnp.zeros(16, jnp.int32)
                for e in range(E):
                    m = v == e
                    mi = jnp.where(m, 1, 0)
                    excl = plsc.cumsum(mi) - mi
                    b_e = jnp.sum(jnp.where(iota == e, base, 0))
                    pos = pos + mi * (b_e + excl)
                    c_e = plsc.all_reduce_population_count(m)
                    base = base + jnp.where(iota == e, c_e, 0)
                pos_ref[pl.ds(j * 16, 16)] = pos
            pltpu.sync_copy(pos_ref, ip_hbm.at[pl.ds(wid * APT2 + q * 64, 64)])
            lds[q].wait()
            scat[q] = pltpu.async_copy(rows[q], sx_hbm.at[pos_ref], sems[q])
        scat[0].wait()
        scat[1].wait()

    return routing(x, tkT)


def _gmm_body(be_ref, xs_ref, wg_ref, wu_ref, wd_ref, y_ref):
    xlo, xhi = _unpack_bf16_pair(xs_ref[...])
    wg = wg_ref[0]
    wu = wu_ref[0]
    g = (jnp.dot(xlo, wg[0:H2], preferred_element_type=jnp.float32)
         + jnp.dot(xhi, wg[H2:HIDDEN], preferred_element_type=jnp.float32))
    u = (jnp.dot(xlo, wu[0:H2], preferred_element_type=jnp.float32)
         + jnp.dot(xhi, wu[H2:HIDDEN], preferred_element_type=jnp.float32))
    h = g * jax.nn.sigmoid(g) * u
    y = jnp.dot(h, wd_ref[0], preferred_element_type=jnp.float32)
    y_ref[...] = _pack_bf16_pair(y[:, 0:H2], y[:, H2:HIDDEN])


def _gmm(blk_exp, sorted_x, weg, weu, wed):
    grid_spec = pltpu.PrefetchScalarGridSpec(
        num_scalar_prefetch=1,
        grid=(NBLK,),
        in_specs=[
            pl.BlockSpec((BLK, H2), lambda b, be: (b, 0)),
            pl.BlockSpec((1, HIDDEN, MOE_FFN), lambda b, be: (be[b], 0, 0)),
            pl.BlockSpec((1, HIDDEN, MOE_FFN), lambda b, be: (be[b], 0, 0)),
            pl.BlockSpec((1, MOE_FFN, HIDDEN), lambda b, be: (be[b], 0, 0)),
        ],
        out_specs=pl.BlockSpec((BLK, H2), lambda b, be: (b, 0)),
    )
    return pl.pallas_call(
        _gmm_body,
        grid_spec=grid_spec,
        out_shape=jax.ShapeDtypeStruct((NPAD, H2), jnp.int32),
    )(blk_exp, sorted_x, weg, weu, wed)


def _gather_sc(y_out, ip):
    mesh = plsc.VectorSubcoreMesh(
        core_axis_name="c", subcore_axis_name="s", num_cores=2)

    @functools.partial(
        pl.kernel,
        out_type=jax.ShapeDtypeStruct((A, H2), jnp.int32),
        mesh=mesh,
        scratch_types=[
            pltpu.VMEM((2, 64), jnp.int32),
            pltpu.VMEM((64, H2), jnp.int32),
            pltpu.VMEM((64, H2), jnp.int32),
            pltpu.SemaphoreType.DMA,
            pltpu.SemaphoreType.DMA,
        ],
        compiler_params=pltpu.CompilerParams(needs_layout_passes=False),
    )
    def gatherk(y_hbm, ip_hbm, yg_hbm, idx_v, gbuf0_v, gbuf1_v, sem, sem2):
        wid = lax.axis_index("s") * 2 + lax.axis_index("c")
        base = wid * 128
        gbufs = [gbuf0_v, gbuf1_v]
        sems = [sem, sem2]
        gets = [None, None]
        for half in range(2):
            idx_ref = idx_v.at[half]
            pltpu.sync_copy(ip_hbm.at[pl.ds(base + half * 64, 64)], idx_ref)
            gets[half] = pltpu.async_copy(y_hbm.at[idx_ref], gbufs[half],
                                          sems[half])
        for half in range(2):
            gets[half].wait()
            pltpu.sync_copy(gbufs[half], yg_hbm.at[pl.ds(base + half * 64, 64)])

    return gatherk(y_out, ip)


def _combine_body(sh_ref, w_ref, y0_ref, y1_ref, out_ref):
    w = w_ref[...]
    w0 = w[:, 0:1]
    w1 = w[:, 1:2]
    y0lo, y0hi = _unpack_bf16_pair(y0_ref[...])
    y1lo, y1hi = _unpack_bf16_pair(y1_ref[...])
    out_ref[:, 0:H2] = sh_ref[:, 0:H2] + w0 * y0lo + w1 * y1lo
    out_ref[:, H2:HIDDEN] = sh_ref[:, H2:HIDDEN] + w0 * y0hi + w1 * y1hi


def _combine(shared, w2, yg):
    n = T // TOK_BLK
    return pl.pallas_call(
        _combine_body,
        grid=(n,),
        in_specs=[
            pl.BlockSpec((TOK_BLK, HIDDEN), lambda t: (t, 0)),
            pl.BlockSpec((TOK_BLK, TOP_K), lambda t: (t, 0)),
            pl.BlockSpec((TOK_BLK, H2), lambda t: (t, 0)),
            pl.BlockSpec((TOK_BLK, H2), lambda t, n=n: (t + n, 0)),
        ],
        out_specs=pl.BlockSpec((TOK_BLK, HIDDEN), lambda t: (t, 0)),
        out_shape=jax.ShapeDtypeStruct((T, HIDDEN), jnp.float32),
    )(shared, w2, yg, yg)


def kernel(hidden_states, W_router, Ws_gate, Ws_up, Ws_down,
           We_gate, We_up, We_down):
    orig_shape = hidden_states.shape
    x = hidden_states.reshape(-1, HIDDEN)

    tk2, w2, xpack = _router(x, W_router)
    tkT = jnp.transpose(tk2).reshape(A)

    sorted_x, ip, blk_exp = _routing_sc(xpack, tkT)
    y_out = _gmm(blk_exp, sorted_x, We_gate, We_up, We_down)
    yg = _gather_sc(y_out, ip)
    out = _shared_combine(x, Ws_gate, Ws_up, Ws_down, w2, yg)
    return out.reshape(orig_shape)


# speedup vs baseline: 1.3905x; 1.0225x over previous
import functools

import jax
import jax.numpy as jnp
from jax import lax
from jax.experimental import pallas as pl
from jax.experimental.pallas import tpu as pltpu
from jax.experimental.pallas import tpu_sc as plsc

HIDDEN = 1024
FFN = 2048
MOE_FFN = 512
E = 8
TOP_K = 2
T = 2048
A = T * TOP_K
BLK = 256
BLK_SHIFT = 8
NPAD = A + E * BLK
NBLK = NPAD // BLK
TOK_BLK = 512

NW = 16
APT = A // NW


H2 = HIDDEN // 2
_HI_MASK = -65536


def _pack_bf16_pair(lo_f32, hi_f32):
    lo = lax.bitcast_convert_type(
        lo_f32.astype(jnp.bfloat16).astype(jnp.float32), jnp.int32)
    hi = lax.bitcast_convert_type(
        hi_f32.astype(jnp.bfloat16).astype(jnp.float32), jnp.int32)
    return lax.shift_right_logical(lo, 16) | (hi & _HI_MASK)


def _unpack_bf16_pair(words):
    lo = lax.bitcast_convert_type(words << 16, jnp.float32)
    hi = lax.bitcast_convert_type(words & _HI_MASK, jnp.float32)
    return lo, hi


def _router_body(x_ref, wr_ref, tk_ref, w_ref, xp_ref):
    logits = jnp.dot(x_ref[...], wr_ref[...], preferred_element_type=jnp.float32)
    iota = lax.broadcasted_iota(jnp.int32, logits.shape, 1)
    v1 = jnp.max(logits, axis=1, keepdims=True)
    idx1 = jnp.min(jnp.where(logits == v1, iota, E), axis=1, keepdims=True)
    sel1 = iota == idx1
    masked = jnp.where(sel1, -jnp.inf, logits)
    v2 = jnp.max(masked, axis=1, keepdims=True)
    idx2 = jnp.min(jnp.where(masked == v2, iota, E), axis=1, keepdims=True)
    e2 = jnp.exp(v2 - v1)
    w1 = 1.0 / (1.0 + e2)
    w2 = 1.0 - w1
    tk_ref[...] = jnp.concatenate([idx1, idx2], axis=1)
    w_ref[...] = jnp.concatenate([w1, w2], axis=1)
    xp_ref[...] = _pack_bf16_pair(x_ref[:, 0:H2], x_ref[:, H2:HIDDEN])


def _router(x, wr):
    n = T // TOK_BLK
    return pl.pallas_call(
        _router_body,
        grid=(n,),
        in_specs=[
            pl.BlockSpec((TOK_BLK, HIDDEN), lambda t: (t, 0)),
            pl.BlockSpec((HIDDEN, E), lambda t: (0, 0)),
        ],
        out_specs=[
            pl.BlockSpec((TOK_BLK, TOP_K), lambda t: (t, 0)),
            pl.BlockSpec((TOK_BLK, TOP_K), lambda t: (t, 0)),
            pl.BlockSpec((TOK_BLK, H2), lambda t: (t, 0)),
        ],
        out_shape=[
            jax.ShapeDtypeStruct((T, TOP_K), jnp.int32),
            jax.ShapeDtypeStruct((T, TOP_K), jnp.float32),
            jax.ShapeDtypeStruct((T, H2), jnp.int32),
        ],
    )(x, wr)


def _shared_combine_body(x_ref, wg_ref, wu_ref, wd_ref, w_ref, y0_ref, y1_ref,
                         out_ref):
    x = x_ref[...]
    g = jnp.dot(x, wg_ref[...], preferred_element_type=jnp.float32)
    u = jnp.dot(x, wu_ref[...], preferred_element_type=jnp.float32)
    h = g * jax.nn.sigmoid(g) * u
    sh = jnp.dot(h, wd_ref[...], preferred_element_type=jnp.float32)
    w = w_ref[...]
    w0 = w[:, 0:1]
    w1 = w[:, 1:2]
    y0lo, y0hi = _unpack_bf16_pair(y0_ref[...])
    y1lo, y1hi = _unpack_bf16_pair(y1_ref[...])
    out_ref[:, 0:H2] = sh[:, 0:H2] + w0 * y0lo + w1 * y1lo
    out_ref[:, H2:HIDDEN] = sh[:, H2:HIDDEN] + w0 * y0hi + w1 * y1hi


def _shared_combine(x, wg, wu, wd, w2, yg):
    n = T // TOK_BLK
    return pl.pallas_call(
        _shared_combine_body,
        grid=(n,),
        in_specs=[
            pl.BlockSpec((TOK_BLK, HIDDEN), lambda t: (t, 0)),
            pl.BlockSpec((HIDDEN, FFN), lambda t: (0, 0)),
            pl.BlockSpec((HIDDEN, FFN), lambda t: (0, 0)),
            pl.BlockSpec((FFN, HIDDEN), lambda t: (0, 0)),
            pl.BlockSpec((TOK_BLK, TOP_K), lambda t: (t, 0)),
            pl.BlockSpec((TOK_BLK, H2), lambda t: (t, 0)),
            pl.BlockSpec((TOK_BLK, H2), lambda t, n=n: (t + n, 0)),
        ],
        out_specs=pl.BlockSpec((TOK_BLK, HIDDEN), lambda t: (t, 0)),
        out_shape=jax.ShapeDtypeStruct((T, HIDDEN), jnp.float32),
    )(x, wg, wu, wd, w2, yg, yg)


NW2 = 32
APT2 = A // NW2
VPT = APT2 // 16
NVREG = A // 16


def _routing_sc(x, tkT):
    mesh = plsc.VectorSubcoreMesh(
        core_axis_name="c", subcore_axis_name="s", num_cores=2)

    @functools.partial(
        pl.kernel,
        out_type=[
            jax.ShapeDtypeStruct((NPAD, H2), jnp.int32),
            jax.ShapeDtypeStruct((A,), jnp.int32),
            jax.ShapeDtypeStruct((48,), jnp.int32),
        ],
        mesh=mesh,
        scratch_types=[
            pltpu.VMEM((A,), jnp.int32),
            pltpu.VMEM((2, 64), jnp.int32),
            pltpu.VMEM((64, H2), jnp.int32),
            pltpu.VMEM((64, H2), jnp.int32),
            pltpu.VMEM((16,), jnp.int32),
            pltpu.SemaphoreType.DMA,
            pltpu.SemaphoreType.DMA,
        ],
        compiler_params=pltpu.CompilerParams(needs_layout_passes=False),
    )
    def routing(xb_hbm, tkT_hbm, sx_hbm, ip_hbm, be_hbm,
                eids_v, pos_v, rows0_v, rows1_v, stage_v, sem, sem2):
        wid = lax.axis_index("s") * 2 + lax.axis_index("c")
        iota = lax.iota(jnp.int32, 16)

        k_tok0 = lax.rem(wid, NW2 // TOP_K) * APT2
        ld0 = pltpu.async_copy(xb_hbm.at[pl.ds(k_tok0, 64)], rows0_v, sem)
        ld1 = pltpu.async_copy(xb_hbm.at[pl.ds(k_tok0 + 64, 64)], rows1_v, sem2)

        pltpu.sync_copy(tkT_hbm, eids_v)

        def scan_body(jj, carry):
            cnt, pre = carry
            pre = jnp.where(jj == wid * VPT, cnt, pre)
            v = eids_v[pl.ds(jj * 16, 16)]
            for e in range(E):
                ce = plsc.all_reduce_population_count(v == e)
                cnt = cnt + jnp.where(iota == e, ce, 0)
            return cnt, pre

        cnt, pre = lax.fori_loop(
            0, NVREG, scan_body,
            (jnp.zeros(16, jnp.int32), jnp.zeros(16, jnp.int32)))
        tot = cnt
        padded = ((tot + (BLK - 1)) >> BLK_SHIFT) << BLK_SHIFT
        pstart = plsc.cumsum(padded) - padded
        base = pstart + pre

        @pl.when(wid == 0)
        def _():
            sblk = pstart >> BLK_SHIFT
            eblk = sblk + (padded >> BLK_SHIFT)
            for bj in range(3):
                bvec = bj * 16 + iota
                be = jnp.zeros(16, jnp.int32)
                for e in range(1, E):
                    s_e = jnp.sum(jnp.where(iota == e, sblk, 0))
                    e_e = jnp.sum(jnp.where(iota == e, eblk, 0))
                    be = be + jnp.where((bvec >= s_e) & (bvec < e_e), e, 0)
                stage_v[...] = be
                pltpu.sync_copy(stage_v, be_hbm.at[pl.ds(bj * 16, 16)])

        rows = [rows0_v, rows1_v]
        lds = [ld0, ld1]
        sems = [sem, sem2]
        scat = [None, None]
        for q in range(2):
            pos_ref = pos_v.at[q]
            for j in range(4):
                v = eids_v[pl.ds(wid * APT2 + q * 64 + j * 16, 16)]
                pos = jnp.zeros(16, jnp.int32)
                for e in range(E):
                    m = v == e
                    mi = jnp.where(m, 1, 0)
                    excl = plsc.cumsum(mi) - mi
                    b_e = jnp.sum(jnp.where(iota == e, base, 0))
                    pos = pos + mi * (b_e + excl)
                    c_e = plsc.all_reduce_population_count(m)
                    base = base + jnp.where(iota == e, c_e, 0)
                pos_ref[pl.ds(j * 16, 16)] = pos
            pltpu.sync_copy(pos_ref, ip_hbm.at[pl.ds(wid * APT2 + q * 64, 64)])
            lds[q].wait()
            scat[q] = pltpu.async_copy(rows[q], sx_hbm.at[pos_ref], sems[q])
        scat[0].wait()
        scat[1].wait()

    return routing(x, tkT)


def _gmm_body(be_ref, xs_ref, wg_ref, wu_ref, wd_ref, y_ref):
    xlo, xhi = _unpack_bf16_pair(xs_ref[...])
    wg = wg_ref[0]
    wu = wu_ref[0]
    g = (jnp.dot(xlo, wg[0:H2], preferred_element_type=jnp.float32)
         + jnp.dot(xhi, wg[H2:HIDDEN], preferred_element_type=jnp.float32))
    u = (jnp.dot(xlo, wu[0:H2], preferred_element_type=jnp.float32)
         + jnp.dot(xhi, wu[H2:HIDDEN], preferred_element_type=jnp.float32))
    h = g * jax.nn.sigmoid(g) * u
    y = jnp.dot(h, wd_ref[0], preferred_element_type=jnp.float32)
    y_ref[...] = _pack_bf16_pair(y[:, 0:H2], y[:, H2:HIDDEN])


def _gmm(blk_exp, sorted_x, weg, weu, wed):
    grid_spec = pltpu.PrefetchScalarGridSpec(
        num_scalar_prefetch=1,
        grid=(NBLK,),
        in_specs=[
            pl.BlockSpec((BLK, H2), lambda b, be: (b, 0)),
            pl.BlockSpec((1, HIDDEN, MOE_FFN), lambda b, be: (be[b], 0, 0)),
            pl.BlockSpec((1, HIDDEN, MOE_FFN), lambda b, be: (be[b], 0, 0)),
            pl.BlockSpec((1, MOE_FFN, HIDDEN), lambda b, be: (be[b], 0, 0)),
        ],
        out_specs=pl.BlockSpec((BLK, H2), lambda b, be: (b, 0)),
    )
    return pl.pallas_call(
        _gmm_body,
        grid_spec=grid_spec,
        out_shape=jax.ShapeDtypeStruct((NPAD, H2), jnp.int32),
    )(blk_exp, sorted_x, weg, weu, wed)


def _gather_sc(y_out, ip):
    mesh = plsc.VectorSubcoreMesh(
        core_axis_name="c", subcore_axis_name="s", num_cores=2)

    @functools.partial(
        pl.kernel,
        out_type=jax.ShapeDtypeStruct((A, H2), jnp.int32),
        mesh=mesh,
        scratch_types=[
            pltpu.VMEM((2, 64), jnp.int32),
            pltpu.VMEM((64, H2), jnp.int32),
            pltpu.VMEM((64, H2), jnp.int32),
            pltpu.SemaphoreType.DMA,
            pltpu.SemaphoreType.DMA,
        ],
        compiler_params=pltpu.CompilerParams(needs_layout_passes=False),
    )
    def gatherk(y_hbm, ip_hbm, yg_hbm, idx_v, gbuf0_v, gbuf1_v, sem, sem2):
        wid = lax.axis_index("s") * 2 + lax.axis_index("c")
        base = wid * 128
        gbufs = [gbuf0_v, gbuf1_v]
        sems = [sem, sem2]
        gets = [None, None]
        for half in range(2):
            idx_ref = idx_v.at[half]
            pltpu.sync_copy(ip_hbm.at[pl.ds(base + half * 64, 64)], idx_ref)
            gets[half] = pltpu.async_copy(y_hbm.at[idx_ref], gbufs[half],
                                          sems[half])
        for half in range(2):
            gets[half].wait()
            pltpu.sync_copy(gbufs[half], yg_hbm.at[pl.ds(base + half * 64, 64)])

    return gatherk(y_out, ip)


def _combine_body(sh_ref, w_ref, y0_ref, y1_ref, out_ref):
    w = w_ref[...]
    w0 = w[:, 0:1]
    w1 = w[:, 1:2]
    y0lo, y0hi = _unpack_bf16_pair(y0_ref[...])
    y1lo, y1hi = _unpack_bf16_pair(y1_ref[...])
    out_ref[:, 0:H2] = sh_ref[:, 0:H2] + w0 * y0lo + w1 * y1lo
    out_ref[:, H2:HIDDEN] = sh_ref[:, H2:HIDDEN] + w0 * y0hi + w1 * y1hi


def _combine(shared, w2, yg):
    n = T // TOK_BLK
    return pl.pallas_call(
        _combine_body,
        grid=(n,),
        in_specs=[
            pl.BlockSpec((TOK_BLK, HIDDEN), lambda t: (t, 0)),
            pl.BlockSpec((TOK_BLK, TOP_K), lambda t: (t, 0)),
            pl.BlockSpec((TOK_BLK, H2), lambda t: (t, 0)),
            pl.BlockSpec((TOK_BLK, H2), lambda t, n=n: (t + n, 0)),
        ],
        out_specs=pl.BlockSpec((TOK_BLK, HIDDEN), lambda t: (t, 0)),
        out_shape=jax.ShapeDtypeStruct((T, HIDDEN), jnp.float32),
    )(shared, w2, yg, yg)


def kernel(hidden_states, W_router, Ws_gate, Ws_up, Ws_down,
           We_gate, We_up, We_down):
    orig_shape = hidden_states.shape
    x = hidden_states.reshape(-1, HIDDEN)

    tk2, w2, xpack = _router(x, W_router)
    tkT = jnp.transpose(tk2).reshape(A)

    sorted_x, ip, blk_exp = _routing_sc(xpack, tkT)
    y_out = _gmm(blk_exp, sorted_x, We_gate, We_up, We_down)
    yg = _gather_sc(y_out, ip)
    out = _shared_combine(x, Ws_gate, Ws_up, Ws_down, w2, yg)
    return out.reshape(orig_shape)
